# baseline (device time: 349521 ns/iter reference)
import jax
import jax.numpy as jnp
from jax import lax
from jax.experimental import pallas as pl
from jax.experimental.pallas import tpu as pltpu

N_DEV = 8


def kernel(x, w_mat):
    m, k_per = x.shape
    _, n = w_mat.shape
    m_chunk = m // N_DEV

    def body(x_ref, w_ref, out_ref, comm_ref, send_sems, recv_sems):
        my = lax.axis_index("i")
        left = lax.rem(my + N_DEV - 1, N_DEV)
        right = lax.rem(my + 1, N_DEV)

        barrier_sem = pltpu.get_barrier_semaphore()
        for nbr in (left, right):
            pl.semaphore_signal(
                barrier_sem, inc=1,
                device_id=(nbr,), device_id_type=pl.DeviceIdType.MESH,
            )
        pl.semaphore_wait(barrier_sem, 2)

        def partial_chunk(c):
            rows = x_ref[pl.ds(c * m_chunk, m_chunk), :]
            return jnp.dot(rows, w_ref[:, :], preferred_element_type=jnp.float32)

        comm_ref[0, :, :] = partial_chunk(left)

        for h in range(N_DEV - 1):
            send_slot = h % 2
            recv_slot = (h + 1) % 2
            rdma = pltpu.make_async_remote_copy(
                src_ref=comm_ref.at[send_slot],
                dst_ref=comm_ref.at[recv_slot],
                send_sem=send_sems.at[send_slot],
                recv_sem=recv_sems.at[recv_slot],
                device_id=(right,),
                device_id_type=pl.DeviceIdType.MESH,
            )
            rdma.start()
            rdma.wait()

            c = lax.rem(my - h - 2 + 2 * N_DEV, N_DEV)
            if h < N_DEV - 2:
                comm_ref[recv_slot, :, :] = comm_ref[recv_slot, :, :] + partial_chunk(c)
            else:
                out_ref[:, :] = comm_ref[recv_slot, :, :] + partial_chunk(c)

    return pl.pallas_call(
        body,
        out_shape=jax.ShapeDtypeStruct((m_chunk, n), jnp.float32),
        in_specs=[
            pl.BlockSpec(memory_space=pltpu.VMEM),
            pl.BlockSpec(memory_space=pltpu.VMEM),
        ],
        out_specs=pl.BlockSpec(memory_space=pltpu.VMEM),
        scratch_shapes=[
            pltpu.VMEM((2, m_chunk, n), jnp.float32),
            pltpu.SemaphoreType.DMA((2,)),
            pltpu.SemaphoreType.DMA((2,)),
        ],
        compiler_params=pltpu.CompilerParams(collective_id=0),
    )(x, w_mat)


# device time: 195277 ns/iter; 1.7899x vs baseline; 1.7899x over previous
import jax
import jax.numpy as jnp
from jax import lax
from jax.experimental import pallas as pl
from jax.experimental.pallas import tpu as pltpu

N_DEV = 8


def kernel(x, w_mat):
    m, k_per = x.shape
    _, n = w_mat.shape
    m_chunk = m // N_DEV
    n_half = n // 2

    def body(x_ref, w_ref, out_ref, cw_ref, ccw_ref,
             cw_send_sems, cw_recv_sems, ccw_send_sems, ccw_recv_sems):
        my = lax.axis_index("i")
        left = lax.rem(my + N_DEV - 1, N_DEV)
        right = lax.rem(my + 1, N_DEV)

        barrier_sem = pltpu.get_barrier_semaphore()
        for nbr in (left, right):
            pl.semaphore_signal(
                barrier_sem, inc=1,
                device_id=(nbr,), device_id_type=pl.DeviceIdType.MESH,
            )
        pl.semaphore_wait(barrier_sem, 2)

        def partial_half(c, lo):
            rows = x_ref[pl.ds(c * m_chunk, m_chunk), :]
            return jnp.dot(rows, w_ref[:, lo:lo + n_half],
                           preferred_element_type=jnp.float32)

        cw_ref[0, :, :] = partial_half(left, 0)
        ccw_ref[0, :, :] = partial_half(right, n_half)

        for h in range(N_DEV - 1):
            s = h % 2
            r = (h + 1) % 2
            cw = pltpu.make_async_remote_copy(
                src_ref=cw_ref.at[s], dst_ref=cw_ref.at[r],
                send_sem=cw_send_sems.at[s], recv_sem=cw_recv_sems.at[r],
                device_id=(right,), device_id_type=pl.DeviceIdType.MESH,
            )
            ccw = pltpu.make_async_remote_copy(
                src_ref=ccw_ref.at[s], dst_ref=ccw_ref.at[r],
                send_sem=ccw_send_sems.at[s], recv_sem=ccw_recv_sems.at[r],
                device_id=(left,), device_id_type=pl.DeviceIdType.MESH,
            )
            cw.start()
            ccw.start()
            cw.wait()
            ccw.wait()

            c_cw = lax.rem(my - h - 2 + 2 * N_DEV, N_DEV)
            c_ccw = lax.rem(my + h + 2, N_DEV)
            if h < N_DEV - 2:
                cw_ref[r, :, :] = cw_ref[r, :, :] + partial_half(c_cw, 0)
                ccw_ref[r, :, :] = ccw_ref[r, :, :] + partial_half(c_ccw, n_half)
            else:
                out_ref[:, :n_half] = cw_ref[r, :, :] + partial_half(c_cw, 0)
                out_ref[:, n_half:] = ccw_ref[r, :, :] + partial_half(c_ccw, n_half)

    return pl.pallas_call(
        body,
        out_shape=jax.ShapeDtypeStruct((m_chunk, n), jnp.float32),
        in_specs=[
            pl.BlockSpec(memory_space=pltpu.VMEM),
            pl.BlockSpec(memory_space=pltpu.VMEM),
        ],
        out_specs=pl.BlockSpec(memory_space=pltpu.VMEM),
        scratch_shapes=[
            pltpu.VMEM((2, m_chunk, n_half), jnp.float32),
            pltpu.VMEM((2, m_chunk, n_half), jnp.float32),
            pltpu.SemaphoreType.DMA((2,)),
            pltpu.SemaphoreType.DMA((2,)),
            pltpu.SemaphoreType.DMA((2,)),
            pltpu.SemaphoreType.DMA((2,)),
        ],
        compiler_params=pltpu.CompilerParams(collective_id=0),
    )(x, w_mat)


# device time: 173389 ns/iter; 2.0158x vs baseline; 1.1262x over previous
import jax
import jax.numpy as jnp
from jax import lax
from jax.experimental import pallas as pl
from jax.experimental.pallas import tpu as pltpu

N_DEV = 8
N_SEG = 4
N_SLOT = 3


def kernel(x, w_mat):
    m, k_per = x.shape
    _, n = w_mat.shape
    m_chunk = m // N_DEV
    n_seg = n // N_SEG

    def body(x_ref, w_ref, out_ref, b0, b1, b2, b3, tmp_ref, *sems):
        my = lax.axis_index("i")
        left = lax.rem(my + N_DEV - 1, N_DEV)
        right = lax.rem(my + 1, N_DEV)

        streams = [(b0, 0, True), (b1, 1, True), (b2, 2, False), (b3, 3, False)]

        barrier_sem = pltpu.get_barrier_semaphore()
        for nbr in (left, right):
            pl.semaphore_signal(
                barrier_sem, inc=1,
                device_id=(nbr,), device_id_type=pl.DeviceIdType.MESH,
            )
        pl.semaphore_wait(barrier_sem, 2)

        def partial_seg(c, i):
            rows = x_ref[pl.ds(c * m_chunk, m_chunk), :]
            return jnp.dot(rows, w_ref[:, i * n_seg:(i + 1) * n_seg],
                           preferred_element_type=jnp.float32)

        def recv_chunk(h, cw):
            if cw:
                return lax.rem(my - h - 2 + 2 * N_DEV, N_DEV)
            return lax.rem(my + h + 2, N_DEV)

        def make_rdma(buf, i, cw, h):
            return pltpu.make_async_remote_copy(
                src_ref=buf.at[h % N_SLOT],
                dst_ref=buf.at[(h + 1) % N_SLOT],
                send_sem=sems[2 * i].at[h % N_SLOT],
                recv_sem=sems[2 * i + 1].at[(h + 1) % N_SLOT],
                device_id=(right if cw else left,),
                device_id_type=pl.DeviceIdType.MESH,
            )

        for buf, i, cw in streams:
            buf[0, :, :] = partial_seg(left if cw else right, i)
            make_rdma(buf, i, cw, 0).start()

        for h in range(N_DEV - 1):
            for buf, i, cw in streams:
                tmp_ref[i, :, :] = partial_seg(recv_chunk(h, cw), i)
            for buf, i, cw in streams:
                r = (h + 1) % N_SLOT
                make_rdma(buf, i, cw, h).wait()
                if h < N_DEV - 2:
                    buf[r, :, :] = buf[r, :, :] + tmp_ref[i, :, :]
                    make_rdma(buf, i, cw, h + 1).start()
                else:
                    out_ref[:, i * n_seg:(i + 1) * n_seg] = (
                        buf[r, :, :] + tmp_ref[i, :, :]
                    )

    return pl.pallas_call(
        body,
        out_shape=jax.ShapeDtypeStruct((m_chunk, n), jnp.float32),
        in_specs=[
            pl.BlockSpec(memory_space=pltpu.VMEM),
            pl.BlockSpec(memory_space=pltpu.VMEM),
        ],
        out_specs=pl.BlockSpec(memory_space=pltpu.VMEM),
        scratch_shapes=[
            pltpu.VMEM((N_SLOT, m_chunk, n // N_SEG), jnp.float32),
            pltpu.VMEM((N_SLOT, m_chunk, n // N_SEG), jnp.float32),
            pltpu.VMEM((N_SLOT, m_chunk, n // N_SEG), jnp.float32),
            pltpu.VMEM((N_SLOT, m_chunk, n // N_SEG), jnp.float32),
            pltpu.VMEM((N_SEG, m_chunk, n // N_SEG), jnp.float32),
        ] + [pltpu.SemaphoreType.DMA((N_SLOT,)) for _ in range(2 * N_SEG)],
        compiler_params=pltpu.CompilerParams(collective_id=0),
    )(x, w_mat)


# device time: 173323 ns/iter; 2.0166x vs baseline; 1.0004x over previous
import jax
import jax.numpy as jnp
from jax import lax
from jax.experimental import pallas as pl
from jax.experimental.pallas import tpu as pltpu

N_DEV = 8
N_SEG = 8
N_SLOT = 3


def kernel(x, w_mat):
    m, k_per = x.shape
    _, n = w_mat.shape
    m_chunk = m // N_DEV
    n_seg = n // N_SEG

    def body(x_ref, w_ref, out_ref, *rest):
        bufs = rest[:N_SEG]
        tmp_ref = rest[N_SEG]
        sems = rest[N_SEG + 1:]

        my = lax.axis_index("i")
        left = lax.rem(my + N_DEV - 1, N_DEV)
        right = lax.rem(my + 1, N_DEV)

        streams = [(bufs[i], i, i < N_SEG // 2) for i in range(N_SEG)]

        barrier_sem = pltpu.get_barrier_semaphore()
        for nbr in (left, right):
            pl.semaphore_signal(
                barrier_sem, inc=1,
                device_id=(nbr,), device_id_type=pl.DeviceIdType.MESH,
            )
        pl.semaphore_wait(barrier_sem, 2)

        def partial_seg(c, i):
            rows = x_ref[pl.ds(c * m_chunk, m_chunk), :]
            return jnp.dot(rows, w_ref[:, i * n_seg:(i + 1) * n_seg],
                           preferred_element_type=jnp.float32)

        def recv_chunk(h, cw):
            if cw:
                return lax.rem(my - h - 2 + 2 * N_DEV, N_DEV)
            return lax.rem(my + h + 2, N_DEV)

        def make_rdma(buf, i, cw, h):
            return pltpu.make_async_remote_copy(
                src_ref=buf.at[h % N_SLOT],
                dst_ref=buf.at[(h + 1) % N_SLOT],
                send_sem=sems[2 * i].at[h % N_SLOT],
                recv_sem=sems[2 * i + 1].at[(h + 1) % N_SLOT],
                device_id=(right if cw else left,),
                device_id_type=pl.DeviceIdType.MESH,
            )

        for buf, i, cw in streams:
            buf[0, :, :] = partial_seg(left if cw else right, i)
            make_rdma(buf, i, cw, 0).start()

        for h in range(N_DEV - 1):
            for buf, i, cw in streams:
                tmp_ref[i, :, :] = partial_seg(recv_chunk(h, cw), i)
            for buf, i, cw in streams:
                r = (h + 1) % N_SLOT
                make_rdma(buf, i, cw, h).wait()
                if h < N_DEV - 2:
                    buf[r, :, :] = buf[r, :, :] + tmp_ref[i, :, :]
                    make_rdma(buf, i, cw, h + 1).start()
                else:
                    out_ref[:, i * n_seg:(i + 1) * n_seg] = (
                        buf[r, :, :] + tmp_ref[i, :, :]
                    )

    return pl.pallas_call(
        body,
        out_shape=jax.ShapeDtypeStruct((m_chunk, n), jnp.float32),
        in_specs=[
            pl.BlockSpec(memory_space=pltpu.VMEM),
            pl.BlockSpec(memory_space=pltpu.VMEM),
        ],
        out_specs=pl.BlockSpec(memory_space=pltpu.VMEM),
        scratch_shapes=[
            pltpu.VMEM((N_SLOT, m_chunk, n // N_SEG), jnp.float32)
            for _ in range(N_SEG)
        ] + [
            pltpu.VMEM((N_SEG, m_chunk, n // N_SEG), jnp.float32),
        ] + [pltpu.SemaphoreType.DMA((N_SLOT,)) for _ in range(2 * N_SEG)],
        compiler_params=pltpu.CompilerParams(collective_id=0),
    )(x, w_mat)


# device time: 142556 ns/iter; 2.4518x vs baseline; 1.2158x over previous
import jax
import jax.numpy as jnp
from jax import lax
from jax.experimental import pallas as pl
from jax.experimental.pallas import tpu as pltpu

N_DEV = 8
U = 1152
V1 = 512
V2 = 384


def kernel(x, w_mat):
    m, k_per = x.shape
    _, n = w_mat.shape
    m_chunk = m // N_DEV

    def body(x_ref, w_ref, out_ref,
             s1cw, s1ccw, s1z_recv,
             z_send, z_recv, s2cw, s2ccw,
             ss1cw, rs1cw, ss1ccw, rs1ccw,
             s1z_ssem, s1z_rsem, z_ssem, z_rsem,
             ss2cw, rs2cw, ss2ccw, rs2ccw):
        my = lax.axis_index("i")
        plane = my // 4
        k = lax.rem(my, 4)
        right = plane * 4 + lax.rem(k + 1, 4)
        left = plane * 4 + lax.rem(k + 3, 4)
        partner = lax.rem(my + 4, N_DEV)

        barrier_sem = pltpu.get_barrier_semaphore()
        for nbr in (right, left, partner):
            pl.semaphore_signal(
                barrier_sem, inc=1,
                device_id=(nbr,), device_id_type=pl.DeviceIdType.MESH,
            )
        pl.semaphore_wait(barrier_sem, 3)

        def partial(chunk_id, lo, width):
            rows = x_ref[pl.ds(chunk_id * m_chunk, m_chunk), :]
            return jnp.dot(rows, w_ref[:, lo:lo + width],
                           preferred_element_type=jnp.float32)

        def ring_rdma(buf, ssem, rsem, h, target):
            return pltpu.make_async_remote_copy(
                src_ref=buf.at[h % 3], dst_ref=buf.at[(h + 1) % 3],
                send_sem=ssem.at[h % 3], recv_sem=rsem.at[(h + 1) % 3],
                device_id=(target,), device_id_type=pl.DeviceIdType.MESH,
            )

        def own_chunk(j):
            return plane * 4 + j
        def other_chunk(j):
            return (1 - plane) * 4 + j

        s1cw[0, :, :] = partial(own_chunk(lax.rem(k + 3, 4)), 0, U)
        ring_rdma(s1cw, ss1cw, rs1cw, 0, right).start()
        s1ccw[0, :, :] = partial(other_chunk(lax.rem(k + 1, 4)), 0, U)
        ring_rdma(s1ccw, ss1ccw, rs1ccw, 0, left).start()

        j_seq = [lax.rem(k + 3, 4), lax.rem(k + 1, 4),
                 lax.rem(k + 2, 4), k]

        def z_rdma(idx):
            return pltpu.make_async_remote_copy(
                src_ref=z_send.at[idx], dst_ref=z_recv.at[idx],
                send_sem=z_ssem.at[idx], recv_sem=z_rsem.at[idx],
                device_id=(partner,), device_id_type=pl.DeviceIdType.MESH,
            )

        for idx in range(4):
            z_send[idx, :, :] = partial(other_chunk(j_seq[idx]), U, n - U)
            z_rdma(idx).start()

        for h in range(3):
            r = (h + 1) % 3
            ring_rdma(s1cw, ss1cw, rs1cw, h, right).wait()
            s1cw[r, :, :] = s1cw[r, :, :] + partial(
                own_chunk(lax.rem(k - h + 6, 4)), 0, U)
            if h < 2:
                ring_rdma(s1cw, ss1cw, rs1cw, h + 1, right).start()
            ring_rdma(s1ccw, ss1ccw, rs1ccw, h, left).wait()
            s1ccw[r, :, :] = s1ccw[r, :, :] + partial(
                other_chunk(lax.rem(k + h + 2, 4)), 0, U)
            if h < 2:
                ring_rdma(s1ccw, ss1ccw, rs1ccw, h + 1, left).start()

        s1z = pltpu.make_async_remote_copy(
            src_ref=s1ccw.at[0], dst_ref=s1z_recv,
            send_sem=s1z_ssem, recv_sem=s1z_rsem,
            device_id=(partner,), device_id_type=pl.DeviceIdType.MESH,
        )
        s1z.start()

        def comb_cw(j, idx):
            return partial(own_chunk(j), U, V1) + z_recv[idx, :, :V1]
        def comb_ccw(j, idx):
            return partial(own_chunk(j), U + V1, V2) + z_recv[idx, :, V1:]

        z_rdma(0).wait_recv()
        s2cw[0, :, :] = comb_cw(lax.rem(k + 3, 4), 0)
        ring_rdma(s2cw, ss2cw, rs2cw, 0, right).start()
        z_rdma(1).wait_recv()
        s2ccw[0, :, :] = comb_ccw(lax.rem(k + 1, 4), 1)
        ring_rdma(s2ccw, ss2ccw, rs2ccw, 0, left).start()

        cw_sched = [(lax.rem(k + 2, 4), 2), (lax.rem(k + 1, 4), 1), (k, 3)]
        ccw_sched = [(lax.rem(k + 2, 4), 2), (lax.rem(k + 3, 4), 0), (k, 3)]
        for h in range(3):
            if h == 0:
                z_rdma(2).wait_recv()
            if h == 2:
                z_rdma(3).wait_recv()
            jc, ic = cw_sched[h]
            jcc, icc = ccw_sched[h]
            r = (h + 1) % 3
            ring_rdma(s2cw, ss2cw, rs2cw, h, right).wait()
            if h < 2:
                s2cw[r, :, :] = s2cw[r, :, :] + comb_cw(jc, ic)
                ring_rdma(s2cw, ss2cw, rs2cw, h + 1, right).start()
            else:
                out_ref[:, U:U + V1] = s2cw[r, :, :] + comb_cw(jc, ic)
            ring_rdma(s2ccw, ss2ccw, rs2ccw, h, left).wait()
            if h < 2:
                s2ccw[r, :, :] = s2ccw[r, :, :] + comb_ccw(jcc, icc)
                ring_rdma(s2ccw, ss2ccw, rs2ccw, h + 1, left).start()
            else:
                out_ref[:, U + V1:] = s2ccw[r, :, :] + comb_ccw(jcc, icc)

        s1z.wait_recv()
        out_ref[:, :U] = s1cw[0, :, :] + s1z_recv[:, :]

        s1z.wait_send()
        for idx in range(4):
            z_rdma(idx).wait_send()

    return pl.pallas_call(
        body,
        out_shape=jax.ShapeDtypeStruct((m_chunk, n), jnp.float32),
        in_specs=[
            pl.BlockSpec(memory_space=pltpu.VMEM),
            pl.BlockSpec(memory_space=pltpu.VMEM),
        ],
        out_specs=pl.BlockSpec(memory_space=pltpu.VMEM),
        scratch_shapes=[
            pltpu.VMEM((3, m_chunk, U), jnp.float32),
            pltpu.VMEM((3, m_chunk, U), jnp.float32),
            pltpu.VMEM((m_chunk, U), jnp.float32),
            pltpu.VMEM((4, m_chunk, n - U), jnp.float32),
            pltpu.VMEM((4, m_chunk, n - U), jnp.float32),
            pltpu.VMEM((3, m_chunk, V1), jnp.float32),
            pltpu.VMEM((3, m_chunk, V2), jnp.float32),
            pltpu.SemaphoreType.DMA((3,)),
            pltpu.SemaphoreType.DMA((3,)),
            pltpu.SemaphoreType.DMA((3,)),
            pltpu.SemaphoreType.DMA((3,)),
            pltpu.SemaphoreType.DMA,
            pltpu.SemaphoreType.DMA,
            pltpu.SemaphoreType.DMA((4,)),
            pltpu.SemaphoreType.DMA((4,)),
            pltpu.SemaphoreType.DMA((3,)),
            pltpu.SemaphoreType.DMA((3,)),
            pltpu.SemaphoreType.DMA((3,)),
            pltpu.SemaphoreType.DMA((3,)),
        ],
        compiler_params=pltpu.CompilerParams(
            collective_id=0,
            vmem_limit_bytes=100 * 1024 * 1024,
        ),
    )(x, w_mat)


# device time: 131718 ns/iter; 2.6536x vs baseline; 1.0823x over previous
import jax
import jax.numpy as jnp
from jax import lax
from jax.experimental import pallas as pl
from jax.experimental.pallas import tpu as pltpu

N_DEV = 8
U = 1024
V1 = 512
V2 = 512


def kernel(x, w_mat):
    m, k_per = x.shape
    _, n = w_mat.shape
    m_chunk = m // N_DEV

    def body(x_ref, w_ref, out_ref,
             s1cw, s1ccw, t1cw, t1ccw, s1z_recv,
             z_send, z_recv, s2cw, s2ccw,
             ss1cw, rs1cw, ss1ccw, rs1ccw,
             s1z_ssem, s1z_rsem, z_ssem, z_rsem,
             ss2cw, rs2cw, ss2ccw, rs2ccw):
        my = lax.axis_index("i")
        plane = my // 4
        k = lax.rem(my, 4)
        right = plane * 4 + lax.rem(k + 1, 4)
        left = plane * 4 + lax.rem(k + 3, 4)
        partner = lax.rem(my + 4, N_DEV)

        barrier_sem = pltpu.get_barrier_semaphore()
        for nbr in (right, left, partner):
            pl.semaphore_signal(
                barrier_sem, inc=1,
                device_id=(nbr,), device_id_type=pl.DeviceIdType.MESH,
            )
        pl.semaphore_wait(barrier_sem, 3)

        def partial(chunk_id, lo, width):
            rows = x_ref[pl.ds(chunk_id * m_chunk, m_chunk), :]
            return jnp.dot(rows, w_ref[:, lo:lo + width],
                           preferred_element_type=jnp.float32)

        def ring_rdma(buf, ssem, rsem, h, target):
            return pltpu.make_async_remote_copy(
                src_ref=buf.at[h % 3], dst_ref=buf.at[(h + 1) % 3],
                send_sem=ssem.at[h % 3], recv_sem=rsem.at[(h + 1) % 3],
                device_id=(target,), device_id_type=pl.DeviceIdType.MESH,
            )

        def own_chunk(j):
            return plane * 4 + j
        def other_chunk(j):
            return (1 - plane) * 4 + j

        s1cw[0, :, :] = partial(own_chunk(lax.rem(k + 3, 4)), 0, U)
        ring_rdma(s1cw, ss1cw, rs1cw, 0, right).start()
        s1ccw[0, :, :] = partial(other_chunk(lax.rem(k + 1, 4)), 0, U)
        ring_rdma(s1ccw, ss1ccw, rs1ccw, 0, left).start()

        j_seq = [lax.rem(k + 3, 4), lax.rem(k + 1, 4),
                 lax.rem(k + 2, 4), k]

        def z_rdma(idx):
            return pltpu.make_async_remote_copy(
                src_ref=z_send.at[idx], dst_ref=z_recv.at[idx],
                send_sem=z_ssem.at[idx], recv_sem=z_rsem.at[idx],
                device_id=(partner,), device_id_type=pl.DeviceIdType.MESH,
            )

        for idx in range(4):
            z_send[idx, :, :] = partial(other_chunk(j_seq[idx]), U, n - U)
            z_rdma(idx).start()

        for h in range(3):
            t1cw[:, :] = partial(own_chunk(lax.rem(k - h + 6, 4)), 0, U)
            t1ccw[:, :] = partial(other_chunk(lax.rem(k + h + 2, 4)), 0, U)
            r = (h + 1) % 3
            ring_rdma(s1cw, ss1cw, rs1cw, h, right).wait()
            s1cw[r, :, :] = s1cw[r, :, :] + t1cw[:, :]
            if h < 2:
                ring_rdma(s1cw, ss1cw, rs1cw, h + 1, right).start()
            ring_rdma(s1ccw, ss1ccw, rs1ccw, h, left).wait()
            s1ccw[r, :, :] = s1ccw[r, :, :] + t1ccw[:, :]
            if h < 2:
                ring_rdma(s1ccw, ss1ccw, rs1ccw, h + 1, left).start()

        s1z = pltpu.make_async_remote_copy(
            src_ref=s1ccw.at[0], dst_ref=s1z_recv,
            send_sem=s1z_ssem, recv_sem=s1z_rsem,
            device_id=(partner,), device_id_type=pl.DeviceIdType.MESH,
        )
        s1z.start()

        def comb_cw(j, idx):
            return partial(own_chunk(j), U, V1) + z_recv[idx, :, :V1]
        def comb_ccw(j, idx):
            return partial(own_chunk(j), U + V1, V2) + z_recv[idx, :, V1:]

        z_rdma(0).wait_recv()
        s2cw[0, :, :] = comb_cw(lax.rem(k + 3, 4), 0)
        ring_rdma(s2cw, ss2cw, rs2cw, 0, right).start()
        z_rdma(1).wait_recv()
        s2ccw[0, :, :] = comb_ccw(lax.rem(k + 1, 4), 1)
        ring_rdma(s2ccw, ss2ccw, rs2ccw, 0, left).start()

        cw_sched = [(lax.rem(k + 2, 4), 2), (lax.rem(k + 1, 4), 1), (k, 3)]
        ccw_sched = [(lax.rem(k + 2, 4), 2), (lax.rem(k + 3, 4), 0), (k, 3)]
        for h in range(3):
            if h == 0:
                z_rdma(2).wait_recv()
            if h == 2:
                z_rdma(3).wait_recv()
            jc, ic = cw_sched[h]
            jcc, icc = ccw_sched[h]
            t1cw[:, :V1] = comb_cw(jc, ic)
            t1ccw[:, :V2] = comb_ccw(jcc, icc)
            r = (h + 1) % 3
            ring_rdma(s2cw, ss2cw, rs2cw, h, right).wait()
            if h < 2:
                s2cw[r, :, :] = s2cw[r, :, :] + t1cw[:, :V1]
                ring_rdma(s2cw, ss2cw, rs2cw, h + 1, right).start()
            else:
                out_ref[:, U:U + V1] = s2cw[r, :, :] + t1cw[:, :V1]
            ring_rdma(s2ccw, ss2ccw, rs2ccw, h, left).wait()
            if h < 2:
                s2ccw[r, :, :] = s2ccw[r, :, :] + t1ccw[:, :V2]
                ring_rdma(s2ccw, ss2ccw, rs2ccw, h + 1, left).start()
            else:
                out_ref[:, U + V1:] = s2ccw[r, :, :] + t1ccw[:, :V2]

        s1z.wait_recv()
        out_ref[:, :U] = s1cw[0, :, :] + s1z_recv[:, :]

        s1z.wait_send()
        for idx in range(4):
            z_rdma(idx).wait_send()

    return pl.pallas_call(
        body,
        out_shape=jax.ShapeDtypeStruct((m_chunk, n), jnp.float32),
        in_specs=[
            pl.BlockSpec(memory_space=pltpu.VMEM),
            pl.BlockSpec(memory_space=pltpu.VMEM),
        ],
        out_specs=pl.BlockSpec(memory_space=pltpu.VMEM),
        scratch_shapes=[
            pltpu.VMEM((3, m_chunk, U), jnp.float32),
            pltpu.VMEM((3, m_chunk, U), jnp.float32),
            pltpu.VMEM((m_chunk, U), jnp.float32),
            pltpu.VMEM((m_chunk, U), jnp.float32),
            pltpu.VMEM((m_chunk, U), jnp.float32),
            pltpu.VMEM((4, m_chunk, n - U), jnp.float32),
            pltpu.VMEM((4, m_chunk, n - U), jnp.float32),
            pltpu.VMEM((3, m_chunk, V1), jnp.float32),
            pltpu.VMEM((3, m_chunk, V2), jnp.float32),
            pltpu.SemaphoreType.DMA((3,)),
            pltpu.SemaphoreType.DMA((3,)),
            pltpu.SemaphoreType.DMA((3,)),
            pltpu.SemaphoreType.DMA((3,)),
            pltpu.SemaphoreType.DMA,
            pltpu.SemaphoreType.DMA,
            pltpu.SemaphoreType.DMA((4,)),
            pltpu.SemaphoreType.DMA((4,)),
            pltpu.SemaphoreType.DMA((3,)),
            pltpu.SemaphoreType.DMA((3,)),
            pltpu.SemaphoreType.DMA((3,)),
            pltpu.SemaphoreType.DMA((3,)),
        ],
        compiler_params=pltpu.CompilerParams(
            collective_id=0,
            vmem_limit_bytes=100 * 1024 * 1024,
        ),
    )(x, w_mat)
